# baseline (speedup 1.0000x reference)
import functools

import jax
import jax.numpy as jnp
from jax import lax
from jax.experimental import pallas as pl
from jax.experimental.pallas import tpu as pltpu
from jax.experimental.pallas import tpu_sc as plsc

N = 10000
E = 320000
F_IN = 128
HID = 16
C = 16

NC = 2
NS = 16
NW = NC * NS
EB = 128
NB = 80
EPT = NB * EB
EPAD = NW * EPT
NSH = 10016
RPT = 640
ZLAST = NSH - (NS - 1) * RPT
CLAST = N - (NS - 1) * RPT


def _mm1_body(x_ref, w_ref, b_ref, tab_ref, r_ref):
    y = jnp.dot(x_ref[...], w_ref[...], preferred_element_type=jnp.float32)
    tab_ref[...] = y[:, :2 * HID]
    r_ref[...] = y[:, 2 * HID:] + b_ref[...]


def _mid_body(acc_ref, cnt_ref, r1b_ref, w_ref, b_ref, tab_ref, r_ref, inv_ref):
    acc = acc_ref[...]
    cnt = cnt_ref[...]
    a = acc[:N] + acc[N:]
    cn = cnt[:N] + cnt[N:]
    inv = 1.0 / jnp.maximum(cn, 1.0)
    hpre = a * inv + r1b_ref[...]
    h = jnp.where(hpre > 0.0, hpre, jnp.exp(hpre) - 1.0)
    y = jnp.dot(h, w_ref[...], preferred_element_type=jnp.float32)
    tab_ref[...] = y[:, :2 * C]
    r_ref[...] = y[:, 2 * C:] + b_ref[...]
    inv_ref[...] = inv


def _final_body(acc_ref, r2b_ref, inv_ref, out_ref):
    acc = acc_ref[...]
    o = (acc[:N] + acc[N:]) * inv_ref[...] + r2b_ref[...]
    m = jnp.max(o, axis=1, keepdims=True)
    e = o - m
    lse = jnp.log(jnp.sum(jnp.exp(e), axis=1, keepdims=True))
    out_ref[...] = e - lse


def _make_agg(width, with_cnt):
    mesh = plsc.VectorSubcoreMesh(
        core_axis_name="c", subcore_axis_name="s", num_cores=NC,
        num_subcores=NS)

    out_type = [jax.ShapeDtypeStruct((NC * N, width), jnp.float32)]
    scratch = {
        "src2d": pltpu.VMEM((NB, EB), jnp.int32),
        "dst2d": pltpu.VMEM((NB, EB), jnp.int32),
        "u2d": pltpu.VMEM((NB, EB), jnp.float32),
        "rows0": pltpu.VMEM((EB, 2 * width), jnp.float32),
        "rows1": pltpu.VMEM((EB, 2 * width), jnp.float32),
        "msg0": pltpu.VMEM((EB, width), jnp.float32),
        "msg1": pltpu.VMEM((EB, width), jnp.float32),
        "zrows": pltpu.VMEM((RPT, width), jnp.float32),
        "tab_sh": pltpu.VMEM_SHARED((N, 2 * width), jnp.float32),
        "acc_sh": pltpu.VMEM_SHARED((NSH, width), jnp.float32),
        "gsem0": pltpu.SemaphoreType.DMA,
        "gsem1": pltpu.SemaphoreType.DMA,
        "ssem0": pltpu.SemaphoreType.DMA,
        "ssem1": pltpu.SemaphoreType.DMA,
    }
    if with_cnt:
        out_type.append(jax.ShapeDtypeStruct((NC * N, width), jnp.float32))
        scratch["ones"] = pltpu.VMEM((EB, width), jnp.float32)
        scratch["cnt_sh"] = pltpu.VMEM_SHARED((NSH, width), jnp.float32)
        scratch["csem0"] = pltpu.SemaphoreType.DMA
        scratch["csem1"] = pltpu.SemaphoreType.DMA

    def body(tab_hbm, src_hbm, dst_hbm, u_hbm, *rest, src2d, dst2d, u2d,
             rows0, rows1, msg0, msg1, zrows, tab_sh, acc_sh, gsem0, gsem1,
             ssem0, ssem1, ones=None, cnt_sh=None, csem0=None, csem1=None):
        if with_cnt:
            acc_hbm, cnt_hbm = rest
        else:
            (acc_hbm,) = rest
            cnt_hbm = None

        cid = lax.axis_index("c")
        sid = lax.axis_index("s")
        wid = cid * NS + sid

        pltpu.sync_copy(src_hbm.at[wid], src2d)
        pltpu.sync_copy(dst_hbm.at[wid], dst2d)
        pltpu.sync_copy(u_hbm.at[wid], u2d)

        zvec = jnp.zeros((width,), jnp.float32)

        def zbody(r, _):
            zrows[r, :] = zvec
            return 0

        lax.fori_loop(0, RPT, zbody, 0)
        if with_cnt:
            onevec = jnp.ones((width,), jnp.float32)

            def obody(r, _):
                ones[r, :] = onevec
                return 0

            lax.fori_loop(0, EB, obody, 0)

        zrow0 = sid * RPT
        last = sid == NS - 1

        @pl.when(jnp.logical_not(last))
        def _():
            pltpu.sync_copy(tab_hbm.at[pl.ds(zrow0, RPT)],
                            tab_sh.at[pl.ds(zrow0, RPT)])
            pltpu.sync_copy(zrows, acc_sh.at[pl.ds(zrow0, RPT)])
            if with_cnt:
                pltpu.sync_copy(zrows, cnt_sh.at[pl.ds(zrow0, RPT)])

        @pl.when(last)
        def _():
            pltpu.sync_copy(tab_hbm.at[pl.ds(zrow0, CLAST)],
                            tab_sh.at[pl.ds(zrow0, CLAST)])
            zpart = zrows.at[pl.ds(0, ZLAST)]
            pltpu.sync_copy(zpart, acc_sh.at[pl.ds(zrow0, ZLAST)])
            if with_cnt:
                pltpu.sync_copy(zpart, cnt_sh.at[pl.ds(zrow0, ZLAST)])

        plsc.subcore_barrier()

        buf = [
            (rows0, msg0, gsem0, ssem0, csem0),
            (rows1, msg1, gsem1, ssem1, csem1),
        ]

        def gather(b, p):
            rows, _, gsem, _, _ = buf[p]
            return pltpu.make_async_copy(tab_sh.at[src2d.at[b]], rows, gsem)

        def msg_scatter(b, p):
            _, msg, _, ssem, _ = buf[p]
            return pltpu.make_async_copy(msg, acc_sh.at[dst2d.at[b]], ssem)

        def cnt_scatter(b, p):
            _, _, _, _, csem = buf[p]
            return pltpu.make_async_copy(ones, cnt_sh.at[dst2d.at[b]], csem)

        def compute(b, p):
            rows, msg, _, _, _ = buf[p]

            @plsc.parallel_loop(0, EB, step=16, unroll=8)
            def ebody(e0):
                uv = u2d[b, pl.ds(e0, 16)]
                for j in range(16):
                    e = e0 + j
                    ub = uv[j]
                    r0 = rows[e, pl.ds(0, width)]
                    dd = rows[e, pl.ds(width, width)]
                    msg[e, :] = r0 + ub * dd

        def half(b, p, first, last_blk):
            gather(b, p).wait()
            if not first:
                msg_scatter(b - 2, p).wait()
                if with_cnt:
                    cnt_scatter(b - 2, p).wait()
            compute(b, p)
            msg_scatter(b, p).start(add=True)
            if with_cnt:
                cnt_scatter(b, p).start(add=True)
            if not last_blk:
                gather(b + 2, p).start()

        gather(0, 0).start()
        gather(1, 1).start()

        half(0, 0, True, False)
        half(1, 1, True, False)

        def blk(j2, _):
            b0 = 2 * j2
            half(b0, 0, False, False)
            half(b0 + 1, 1, False, False)
            return 0

        lax.fori_loop(1, NB // 2 - 1, blk, 0)
        half(NB - 2, 0, False, True)
        half(NB - 1, 1, False, True)
        msg_scatter(NB - 2, 0).wait()
        msg_scatter(NB - 1, 1).wait()
        if with_cnt:
            cnt_scatter(NB - 2, 0).wait()
            cnt_scatter(NB - 1, 1).wait()
        plsc.subcore_barrier()

        crow0 = sid * RPT
        out0 = cid * N + crow0

        @pl.when(jnp.logical_not(last))
        def _():
            pltpu.sync_copy(acc_sh.at[pl.ds(crow0, RPT)],
                            acc_hbm.at[pl.ds(out0, RPT)])
            if with_cnt:
                pltpu.sync_copy(cnt_sh.at[pl.ds(crow0, RPT)],
                                cnt_hbm.at[pl.ds(out0, RPT)])

        @pl.when(last)
        def _():
            pltpu.sync_copy(acc_sh.at[pl.ds(crow0, CLAST)],
                            acc_hbm.at[pl.ds(out0, CLAST)])
            if with_cnt:
                pltpu.sync_copy(cnt_sh.at[pl.ds(crow0, CLAST)],
                                cnt_hbm.at[pl.ds(out0, CLAST)])

    return functools.partial(
        pl.kernel, mesh=mesh, out_type=out_type, scratch_types=scratch,
        compiler_params=pltpu.CompilerParams(use_tc_tiling_on_sc=False))(body)


_agg1 = _make_agg(HID, True)
_agg2 = _make_agg(C, False)


def kernel(x, edge_index, edge_attr, W1, root1, b1, W2, root2, b2):
    ept_real = E // NW
    pad = EPT - ept_real
    src = jnp.concatenate(
        [edge_index[0].reshape(NW, ept_real),
         jnp.zeros((NW, pad), jnp.int32)], axis=1).reshape(NW, NB, EB)
    dummy_dst = jnp.broadcast_to(
        N + (jnp.arange(pad, dtype=jnp.int32) % (NSH - N)), (NW, pad))
    dst = jnp.concatenate(
        [edge_index[1].reshape(NW, ept_real), dummy_dst],
        axis=1).reshape(NW, NB, EB)
    u = jnp.concatenate(
        [edge_attr[:, 0].reshape(NW, ept_real),
         jnp.zeros((NW, pad), jnp.float32)], axis=1).reshape(NW, NB, EB)

    w1c = jnp.concatenate([W1[0], W1[1] - W1[0], root1], axis=1)
    w2c = jnp.concatenate([W2[0], W2[1] - W2[0], root2], axis=1)

    tab1, r1b = pl.pallas_call(
        _mm1_body,
        out_shape=[
            jax.ShapeDtypeStruct((N, 2 * HID), jnp.float32),
            jax.ShapeDtypeStruct((N, HID), jnp.float32),
        ],
    )(x, w1c, b1.reshape(1, HID))

    acc1, cnt = _agg1(tab1, src, dst, u)

    tab2, r2b, inv16 = pl.pallas_call(
        _mid_body,
        out_shape=[
            jax.ShapeDtypeStruct((N, 2 * C), jnp.float32),
            jax.ShapeDtypeStruct((N, C), jnp.float32),
            jax.ShapeDtypeStruct((N, HID), jnp.float32),
        ],
    )(acc1, cnt, r1b, w2c, b2.reshape(1, C))

    (acc2,) = _agg2(tab2, src, dst, u)

    out = pl.pallas_call(
        _final_body,
        out_shape=jax.ShapeDtypeStruct((N, C), jnp.float32),
    )(acc2, r2b, inv16)
    return out


# speedup vs baseline: 1.1111x; 1.1111x over previous
import functools

import jax
import jax.numpy as jnp
from jax import lax
from jax.experimental import pallas as pl
from jax.experimental.pallas import tpu as pltpu
from jax.experimental.pallas import tpu_sc as plsc

N = 10000
E = 320000
F_IN = 128
HID = 16
C = 16

NC = 2
NS = 16
NW = NC * NS
EB = 128
NB = 80
EPT = NB * EB
EPAD = NW * EPT
NSH = 10016
RPT = 640
ZLAST = NSH - (NS - 1) * RPT
CLAST = N - (NS - 1) * RPT


def _mm1_body(x_ref, w_ref, b_ref, tab_ref, r_ref):
    y = jnp.dot(x_ref[...], w_ref[...], preferred_element_type=jnp.float32)
    tab_ref[...] = y[:, :2 * HID]
    r_ref[...] = y[:, 2 * HID:] + b_ref[...]


def _mid_body(acc_ref, r1b_ref, w_ref, b_ref, tab_ref, r_ref, inv_ref):
    v = acc_ref[...]
    a = v[:N, :HID] + v[N:, :HID]
    cn = v[:N, HID:] + v[N:, HID:]
    inv = 1.0 / jnp.maximum(cn, 1.0)
    hpre = a * inv + r1b_ref[...]
    h = jnp.where(hpre > 0.0, hpre, jnp.exp(hpre) - 1.0)
    y = jnp.dot(h, w_ref[...], preferred_element_type=jnp.float32)
    tab_ref[...] = y[:, :2 * C]
    r_ref[...] = y[:, 2 * C:] + b_ref[...]
    inv_ref[...] = inv


def _final_body(acc_ref, r2b_ref, inv_ref, out_ref):
    acc = acc_ref[...]
    o = (acc[:N] + acc[N:]) * inv_ref[...] + r2b_ref[...]
    m = jnp.max(o, axis=1, keepdims=True)
    e = o - m
    lse = jnp.log(jnp.sum(jnp.exp(e), axis=1, keepdims=True))
    out_ref[...] = e - lse


def _make_agg(width, with_cnt):
    mesh = plsc.VectorSubcoreMesh(
        core_axis_name="c", subcore_axis_name="s", num_cores=NC,
        num_subcores=NS)

    sw = 2 * width if with_cnt else width
    out_type = [jax.ShapeDtypeStruct((NC * N, sw), jnp.float32)]
    scratch = {
        "src2d": pltpu.VMEM((NB, EB), jnp.int32),
        "dst2d": pltpu.VMEM((NB, EB), jnp.int32),
        "u2d": pltpu.VMEM((NB, EB), jnp.float32),
        "rows0": pltpu.VMEM((EB, 2 * width), jnp.float32),
        "rows1": pltpu.VMEM((EB, 2 * width), jnp.float32),
        "msg0": pltpu.VMEM((EB, sw), jnp.float32),
        "msg1": pltpu.VMEM((EB, sw), jnp.float32),
        "zrows": pltpu.VMEM((RPT, sw), jnp.float32),
        "tab_sh": pltpu.VMEM_SHARED((N, 2 * width), jnp.float32),
        "acc_sh": pltpu.VMEM_SHARED((NSH, sw), jnp.float32),
        "gsem0": pltpu.SemaphoreType.DMA,
        "gsem1": pltpu.SemaphoreType.DMA,
        "ssem0": pltpu.SemaphoreType.DMA,
        "ssem1": pltpu.SemaphoreType.DMA,
    }

    def body(tab_hbm, src_hbm, dst_hbm, u_hbm, acc_hbm, *, src2d, dst2d, u2d,
             rows0, rows1, msg0, msg1, zrows, tab_sh, acc_sh, gsem0, gsem1,
             ssem0, ssem1):

        cid = lax.axis_index("c")
        sid = lax.axis_index("s")
        wid = cid * NS + sid

        pltpu.sync_copy(src_hbm.at[wid], src2d)
        pltpu.sync_copy(dst_hbm.at[wid], dst2d)
        pltpu.sync_copy(u_hbm.at[wid], u2d)

        zvec = jnp.zeros((16,), jnp.float32)

        def zbody(r, _):
            for q in range(sw // 16):
                zrows[r, pl.ds(16 * q, 16)] = zvec
            return 0

        lax.fori_loop(0, RPT, zbody, 0)
        if with_cnt:
            onevec = jnp.ones((16,), jnp.float32)

            def obody(r, _):
                msg0[r, pl.ds(width, 16)] = onevec
                msg1[r, pl.ds(width, 16)] = onevec
                return 0

            lax.fori_loop(0, EB, obody, 0)

        zrow0 = sid * RPT
        last = sid == NS - 1

        @pl.when(jnp.logical_not(last))
        def _():
            pltpu.sync_copy(tab_hbm.at[pl.ds(zrow0, RPT)],
                            tab_sh.at[pl.ds(zrow0, RPT)])
            pltpu.sync_copy(zrows, acc_sh.at[pl.ds(zrow0, RPT)])

        @pl.when(last)
        def _():
            pltpu.sync_copy(tab_hbm.at[pl.ds(zrow0, CLAST)],
                            tab_sh.at[pl.ds(zrow0, CLAST)])
            zpart = zrows.at[pl.ds(0, ZLAST)]
            pltpu.sync_copy(zpart, acc_sh.at[pl.ds(zrow0, ZLAST)])

        plsc.subcore_barrier()

        buf = [
            (rows0, msg0, gsem0, ssem0),
            (rows1, msg1, gsem1, ssem1),
        ]

        def gather(b, p):
            rows, _, gsem, _ = buf[p]
            return pltpu.make_async_copy(tab_sh.at[src2d.at[b]], rows, gsem)

        def msg_scatter(b, p):
            _, msg, _, ssem = buf[p]
            return pltpu.make_async_copy(msg, acc_sh.at[dst2d.at[b]], ssem)

        def compute(b, p):
            rows, msg, _, _ = buf[p]

            @plsc.parallel_loop(0, EB, step=16, unroll=4)
            def ebody(e0):
                uv = u2d[b, pl.ds(e0, 16)]
                for j in range(16):
                    e = e0 + j
                    ub = uv[j]
                    r0 = rows[e, pl.ds(0, width)]
                    dd = rows[e, pl.ds(width, width)]
                    msg[e, pl.ds(0, width)] = r0 + ub * dd

        def half(b, p, first, last_blk):
            gather(b, p).wait()
            if not first:
                msg_scatter(b - 2, p).wait()
            compute(b, p)
            msg_scatter(b, p).start(add=True)
            if not last_blk:
                gather(b + 2, p).start()

        gather(0, 0).start()
        gather(1, 1).start()

        half(0, 0, True, False)
        half(1, 1, True, False)

        def blk(j2, _):
            b0 = 2 * j2
            half(b0, 0, False, False)
            half(b0 + 1, 1, False, False)
            return 0

        lax.fori_loop(1, NB // 2 - 1, blk, 0)
        half(NB - 2, 0, False, True)
        half(NB - 1, 1, False, True)
        msg_scatter(NB - 2, 0).wait()
        msg_scatter(NB - 1, 1).wait()
        plsc.subcore_barrier()

        crow0 = sid * RPT
        out0 = cid * N + crow0

        @pl.when(jnp.logical_not(last))
        def _():
            pltpu.sync_copy(acc_sh.at[pl.ds(crow0, RPT)],
                            acc_hbm.at[pl.ds(out0, RPT)])

        @pl.when(last)
        def _():
            pltpu.sync_copy(acc_sh.at[pl.ds(crow0, CLAST)],
                            acc_hbm.at[pl.ds(out0, CLAST)])

    return functools.partial(
        pl.kernel, mesh=mesh, out_type=out_type, scratch_types=scratch,
        compiler_params=pltpu.CompilerParams(use_tc_tiling_on_sc=False))(body)


_agg1 = _make_agg(HID, True)
_agg2 = _make_agg(C, False)


def kernel(x, edge_index, edge_attr, W1, root1, b1, W2, root2, b2):
    ept_real = E // NW
    pad = EPT - ept_real
    src = jnp.concatenate(
        [edge_index[0].reshape(NW, ept_real),
         jnp.zeros((NW, pad), jnp.int32)], axis=1).reshape(NW, NB, EB)
    dummy_dst = jnp.broadcast_to(
        N + (jnp.arange(pad, dtype=jnp.int32) % (NSH - N)), (NW, pad))
    dst = jnp.concatenate(
        [edge_index[1].reshape(NW, ept_real), dummy_dst],
        axis=1).reshape(NW, NB, EB)
    u = jnp.concatenate(
        [edge_attr[:, 0].reshape(NW, ept_real),
         jnp.zeros((NW, pad), jnp.float32)], axis=1).reshape(NW, NB, EB)

    w1c = jnp.concatenate([W1[0], W1[1] - W1[0], root1], axis=1)
    w2c = jnp.concatenate([W2[0], W2[1] - W2[0], root2], axis=1)

    tab1, r1b = pl.pallas_call(
        _mm1_body,
        out_shape=[
            jax.ShapeDtypeStruct((N, 2 * HID), jnp.float32),
            jax.ShapeDtypeStruct((N, HID), jnp.float32),
        ],
    )(x, w1c, b1.reshape(1, HID))

    (acc1,) = _agg1(tab1, src, dst, u)

    tab2, r2b, inv16 = pl.pallas_call(
        _mid_body,
        out_shape=[
            jax.ShapeDtypeStruct((N, 2 * C), jnp.float32),
            jax.ShapeDtypeStruct((N, C), jnp.float32),
            jax.ShapeDtypeStruct((N, HID), jnp.float32),
        ],
    )(acc1, r1b, w2c, b2.reshape(1, C))

    (acc2,) = _agg2(tab2, src, dst, u)

    out = pl.pallas_call(
        _final_body,
        out_shape=jax.ShapeDtypeStruct((N, C), jnp.float32),
    )(acc2, r2b, inv16)
    return out
